# baseline (speedup 1.0000x reference)
import jax
import jax.numpy as jnp
from jax import lax
from jax.experimental import pallas as pl
from jax.experimental.pallas import tpu as pltpu
from jax.experimental.pallas import tpu_sc as plsc

A = 256
L = 8
T = 1024
PL = T - 1 - (L - 1)
N = A * PL
NC = 2
NS = 16
NW = NC * NS
LANES = 16
APW = A // NW
GPW = APW * PL
MAGIC = 8257
SHIFT = 23
RMAGIC = 33027
RSHIFT = 22
NROW = 2 * N // 128


def _sc_body(ids_hbm, ts_hbm, ag_hbm, dts_hbm, ids_v, tl_v, ag_v, dts_v):
    wid = lax.axis_index("s") * NC + lax.axis_index("c")
    lane = lax.iota(jnp.int32, LANES)

    pltpu.sync_copy(ids_hbm.at[pl.ds(wid * APW, APW)], ids_v.at[pl.ds(0, APW)])
    pltpu.sync_copy(ts_hbm.at[pl.ds(T - LANES, LANES)], tl_v)
    tbase = tl_v[...][LANES - 1] - PL

    @plsc.parallel_loop(0, GPW // LANES, unroll=2)
    def _agts_loop(v):
        p = v * LANES + lane
        a = (p * MAGIC) >> SHIFT
        k = p - a * PL
        ag_v[pl.ds(v * LANES, LANES)] = plsc.load_gather(ids_v, [a])
        dts_v[pl.ds(v * LANES, LANES)] = tbase + k

    pltpu.sync_copy(ag_v, ag_hbm.at[pl.ds(wid * GPW, GPW)])
    pltpu.sync_copy(dts_v, dts_hbm.at[pl.ds(wid * GPW, GPW)])


def _run_sc(ids, ts):
    mesh = plsc.VectorSubcoreMesh(core_axis_name="c", subcore_axis_name="s",
                                  num_cores=NC, num_subcores=NS)
    f = pl.kernel(
        _sc_body,
        out_type=(
            jax.ShapeDtypeStruct((N,), jnp.int32),
            jax.ShapeDtypeStruct((N,), jnp.int32),
        ),
        mesh=mesh,
        scratch_types=[
            pltpu.VMEM((LANES,), jnp.int32),
            pltpu.VMEM((LANES,), jnp.int32),
            pltpu.VMEM((GPW,), jnp.int32),
            pltpu.VMEM((GPW,), jnp.int32),
        ],
        compiler_params=pltpu.CompilerParams(needs_layout_passes=False),
        name="cv_agents_ts_sc",
    )
    return f(ids, ts)


def _tc_motion_body(ids_ref, lts_ref, posf_ref, velf_ref, out_ref):
    ids = ids_ref[...]
    lts = lts_ref[...]
    posf = posf_ref[...]
    velf = velf_ref[...]

    gsel = 2 * lts + (ids >> 7)
    px = jnp.zeros((1, A), jnp.float32)
    py = jnp.zeros((1, A), jnp.float32)
    vx = jnp.zeros((1, A), jnp.float32)
    vy = jnp.zeros((1, A), jnp.float32)
    for g in range(2 * L):
        m = gsel == g
        pxg = jnp.concatenate([posf[:, g * 256:g * 256 + 128]] * 2, 1)
        pyg = jnp.concatenate([posf[:, g * 256 + 128:g * 256 + 256]] * 2, 1)
        vxg = jnp.concatenate([velf[:, g * 256:g * 256 + 128]] * 2, 1)
        vyg = jnp.concatenate([velf[:, g * 256 + 128:g * 256 + 256]] * 2, 1)
        px = jnp.where(m, pxg, px)
        py = jnp.where(m, pyg, py)
        vx = jnp.where(m, vxg, vx)
        vy = jnp.where(m, vyg, vy)
    tbl = jnp.concatenate([px.T, py.T, vx.T, vy.T], 1)
    tbl_s = jnp.concatenate([tbl[1:], tbl[255:]], 0)

    r_col = lax.broadcasted_iota(jnp.int32, (NROW, 1), 0)
    b = r_col >> 1
    c = r_col & 1
    p0 = b * 128
    a0 = ((b * 16) * RMAGIC) >> RSHIFT
    thr = (a0 + 1) * PL - p0
    kb0 = p0 - a0 * PL + 1

    a_row = lax.broadcasted_iota(jnp.int32, (1, 256), 1)
    oh_row = (a_row == a0).astype(jnp.float32)
    def _split3(x):
        hi = x.astype(jnp.bfloat16).astype(jnp.float32)
        r1 = x - hi
        mid = r1.astype(jnp.bfloat16).astype(jnp.float32)
        return hi, mid, r1 - mid
    h1, m1, l1 = _split3(tbl)
    h2, m2, l2 = _split3(tbl_s)
    rhs = jnp.concatenate([h1, m1, l1, h2, m2, l2], 1)
    y = jnp.dot(oh_row, rhs, preferred_element_type=jnp.float32)
    t1 = y[:, 0:4] + y[:, 4:8] + y[:, 8:12]
    t2 = y[:, 12:16] + y[:, 16:20] + y[:, 20:24]

    cz = c == 0
    p1 = jnp.where(cz, t1[:, 0:1], t1[:, 1:2])
    v1 = jnp.where(cz, t1[:, 2:3], t1[:, 3:4])
    p2 = jnp.where(cz, t2[:, 0:1], t2[:, 1:2])
    v2 = jnp.where(cz, t2[:, 2:3], t2[:, 3:4])

    j_row = lax.broadcasted_iota(jnp.int32, (1, 128), 1)
    s0 = (kb0 + j_row).astype(jnp.float32)
    s1 = s0 - jnp.float32(PL)
    out_ref[...] = jnp.where(j_row < thr, p1 + s0 * v1, p2 + s1 * v2)


def _run_tc(ids2, lts2, posf, velf):
    return pl.pallas_call(
        _tc_motion_body,
        out_shape=jax.ShapeDtypeStruct((NROW, 128), jnp.float32),
        name="cv_motion_tc",
    )(ids2, lts2, posf, velf)


@jax.jit
def _run_all(ids2, lts2, ts1, posf, velf):
    mot = _run_tc(ids2, lts2, posf, velf)
    ag, dts = _run_sc(ids2[0], ts1)
    return mot, ag, dts


def kernel(identities, timesteps, scene_orig, obs_position_sequence,
           obs_velocity_sequence, obs_timestep_sequence, obs_identity_sequence,
           last_obs_positions, last_obs_timesteps, pred_position_sequence,
           pred_velocity_sequence, pred_timestep_sequence, pred_identity_sequence):
    posf = obs_position_sequence[0].reshape(16, 128, 2).transpose(0, 2, 1).reshape(1, -1)
    velf = obs_velocity_sequence[0].reshape(16, 128, 2).transpose(0, 2, 1).reshape(1, -1)
    mot, ag, dts = _run_all(identities, last_obs_timesteps, timesteps[0],
                            posf, velf)
    motion = mot.reshape(2032, 2, 128).transpose(0, 2, 1).reshape(1, N, 2)
    return motion, ag.reshape(1, N), dts


# speedup vs baseline: 1.0118x; 1.0118x over previous
import jax
import jax.numpy as jnp
from jax import lax
from jax.experimental import pallas as pl
from jax.experimental.pallas import tpu as pltpu
from jax.experimental.pallas import tpu_sc as plsc

A = 256
L = 8
T = 1024
PL = T - 1 - (L - 1)
N = A * PL
NC = 2
NS = 16
NW = NC * NS
LANES = 16
APW = A // NW
GPW = APW * PL
MAGIC = 8257
SHIFT = 23
RMAGIC = 33027
RSHIFT = 22
NROW = 2 * N // 128


def _sc_body(ids_hbm, ts_hbm, ag_hbm, dts_hbm, ids_v, tl_v, ag_v, dts_v):
    wid = lax.axis_index("s") * NC + lax.axis_index("c")
    lane = lax.iota(jnp.int32, LANES)

    pltpu.sync_copy(ids_hbm.at[pl.ds(wid * APW, APW)], ids_v.at[pl.ds(0, APW)])
    pltpu.sync_copy(ts_hbm.at[pl.ds(T - LANES, LANES)], tl_v)
    tbase = tl_v[...][LANES - 1] - PL

    @plsc.parallel_loop(0, GPW // LANES, unroll=8)
    def _agts_loop(v):
        p = v * LANES + lane
        a = (p * MAGIC) >> SHIFT
        k = p - a * PL
        ag_v[pl.ds(v * LANES, LANES)] = plsc.load_gather(ids_v, [a])
        dts_v[pl.ds(v * LANES, LANES)] = tbase + k

    pltpu.sync_copy(ag_v, ag_hbm.at[pl.ds(wid * GPW, GPW)])
    pltpu.sync_copy(dts_v, dts_hbm.at[pl.ds(wid * GPW, GPW)])


def _run_sc(ids, ts):
    mesh = plsc.VectorSubcoreMesh(core_axis_name="c", subcore_axis_name="s",
                                  num_cores=NC, num_subcores=NS)
    f = pl.kernel(
        _sc_body,
        out_type=(
            jax.ShapeDtypeStruct((N,), jnp.int32),
            jax.ShapeDtypeStruct((N,), jnp.int32),
        ),
        mesh=mesh,
        scratch_types=[
            pltpu.VMEM((LANES,), jnp.int32),
            pltpu.VMEM((LANES,), jnp.int32),
            pltpu.VMEM((GPW,), jnp.int32),
            pltpu.VMEM((GPW,), jnp.int32),
        ],
        compiler_params=pltpu.CompilerParams(needs_layout_passes=False),
        name="cv_agents_ts_sc",
    )
    return f(ids, ts)


def _tc_motion_body(ids_ref, lts_ref, posf_ref, velf_ref, out_ref):
    ids = ids_ref[...]
    lts = lts_ref[...]
    posf = posf_ref[...]
    velf = velf_ref[...]

    gsel = 2 * lts + (ids >> 7)
    px = jnp.zeros((1, A), jnp.float32)
    py = jnp.zeros((1, A), jnp.float32)
    vx = jnp.zeros((1, A), jnp.float32)
    vy = jnp.zeros((1, A), jnp.float32)
    for g in range(2 * L):
        m = gsel == g
        pxg = jnp.concatenate([posf[:, g * 256:g * 256 + 128]] * 2, 1)
        pyg = jnp.concatenate([posf[:, g * 256 + 128:g * 256 + 256]] * 2, 1)
        vxg = jnp.concatenate([velf[:, g * 256:g * 256 + 128]] * 2, 1)
        vyg = jnp.concatenate([velf[:, g * 256 + 128:g * 256 + 256]] * 2, 1)
        px = jnp.where(m, pxg, px)
        py = jnp.where(m, pyg, py)
        vx = jnp.where(m, vxg, vx)
        vy = jnp.where(m, vyg, vy)
    tbl = jnp.concatenate([px.T, py.T, vx.T, vy.T], 1)
    tbl_s = jnp.concatenate([tbl[1:], tbl[255:]], 0)

    r_col = lax.broadcasted_iota(jnp.int32, (NROW, 1), 0)
    b = r_col >> 1
    c = r_col & 1
    p0 = b * 128
    a0 = ((b * 16) * RMAGIC) >> RSHIFT
    thr = (a0 + 1) * PL - p0
    kb0 = p0 - a0 * PL + 1

    a_row = lax.broadcasted_iota(jnp.int32, (1, 256), 1)
    oh_row = (a_row == a0).astype(jnp.bfloat16)
    def _split3(x):
        hi = x.astype(jnp.bfloat16)
        r1 = x - hi.astype(jnp.float32)
        mid = r1.astype(jnp.bfloat16)
        return hi, mid, (r1 - mid.astype(jnp.float32)).astype(jnp.bfloat16)
    h1, m1, l1 = _split3(tbl)
    h2, m2, l2 = _split3(tbl_s)
    rhs = jnp.concatenate([h1, m1, l1, h2, m2, l2], 1)
    y = jnp.dot(oh_row, rhs, preferred_element_type=jnp.float32)
    t1 = y[:, 0:4] + y[:, 4:8] + y[:, 8:12]
    t2 = y[:, 12:16] + y[:, 16:20] + y[:, 20:24]

    cz = c == 0
    p1 = jnp.where(cz, t1[:, 0:1], t1[:, 1:2])
    v1 = jnp.where(cz, t1[:, 2:3], t1[:, 3:4])
    p2 = jnp.where(cz, t2[:, 0:1], t2[:, 1:2])
    v2 = jnp.where(cz, t2[:, 2:3], t2[:, 3:4])

    j_row = lax.broadcasted_iota(jnp.int32, (1, 128), 1)
    s0 = (kb0 + j_row).astype(jnp.float32)
    s1 = s0 - jnp.float32(PL)
    out_ref[...] = jnp.where(j_row < thr, p1 + s0 * v1, p2 + s1 * v2)


def _run_tc(ids2, lts2, posf, velf):
    return pl.pallas_call(
        _tc_motion_body,
        out_shape=jax.ShapeDtypeStruct((NROW, 128), jnp.float32),
        name="cv_motion_tc",
    )(ids2, lts2, posf, velf)


@jax.jit
def _run_all(ids2, lts2, ts1, posf, velf):
    mot = _run_tc(ids2, lts2, posf, velf)
    ag, dts = _run_sc(ids2[0], ts1)
    return mot, ag, dts


def kernel(identities, timesteps, scene_orig, obs_position_sequence,
           obs_velocity_sequence, obs_timestep_sequence, obs_identity_sequence,
           last_obs_positions, last_obs_timesteps, pred_position_sequence,
           pred_velocity_sequence, pred_timestep_sequence, pred_identity_sequence):
    posf = obs_position_sequence[0].reshape(16, 128, 2).transpose(0, 2, 1).reshape(1, -1)
    velf = obs_velocity_sequence[0].reshape(16, 128, 2).transpose(0, 2, 1).reshape(1, -1)
    mot, ag, dts = _run_all(identities, last_obs_timesteps, timesteps[0],
                            posf, velf)
    motion = mot.reshape(2032, 2, 128).transpose(0, 2, 1).reshape(1, N, 2)
    return motion, ag.reshape(1, N), dts


# speedup vs baseline: 1.0440x; 1.0317x over previous
import jax
import jax.numpy as jnp
from jax import lax
from jax.experimental import pallas as pl
from jax.experimental.pallas import tpu as pltpu
from jax.experimental.pallas import tpu_sc as plsc

A = 256
L = 8
T = 1024
PL = T - 1 - (L - 1)
N = A * PL
NC = 1
NS = 16
NW = NC * NS
LANES = 16
APW = A // NW
GPW = APW * PL
MAGIC = 8257
SHIFT = 23
RMAGIC = 33027
RSHIFT = 22
NROW = 2 * N // 128


def _sc_body(ids_hbm, ts_hbm, ag_hbm, dts_hbm, ids_v, tl_v, ag_v, dts_v):
    wid = lax.axis_index("s") * NC + lax.axis_index("c")
    lane = lax.iota(jnp.int32, LANES)

    pltpu.sync_copy(ids_hbm.at[pl.ds(wid * APW, APW)], ids_v.at[pl.ds(0, APW)])
    pltpu.sync_copy(ts_hbm.at[pl.ds(T - LANES, LANES)], tl_v)
    tbase = tl_v[...][LANES - 1] - PL

    @plsc.parallel_loop(0, GPW // LANES, unroll=8)
    def _agts_loop(v):
        p = v * LANES + lane
        a = (p * MAGIC) >> SHIFT
        k = p - a * PL
        ag_v[pl.ds(v * LANES, LANES)] = plsc.load_gather(ids_v, [a])
        dts_v[pl.ds(v * LANES, LANES)] = tbase + k

    pltpu.sync_copy(ag_v, ag_hbm.at[pl.ds(wid * GPW, GPW)])
    pltpu.sync_copy(dts_v, dts_hbm.at[pl.ds(wid * GPW, GPW)])


def _run_sc(ids, ts):
    mesh = plsc.VectorSubcoreMesh(core_axis_name="c", subcore_axis_name="s",
                                  num_cores=NC, num_subcores=NS)
    f = pl.kernel(
        _sc_body,
        out_type=(
            jax.ShapeDtypeStruct((N,), jnp.int32),
            jax.ShapeDtypeStruct((N,), jnp.int32),
        ),
        mesh=mesh,
        scratch_types=[
            pltpu.VMEM((LANES,), jnp.int32),
            pltpu.VMEM((LANES,), jnp.int32),
            pltpu.VMEM((GPW,), jnp.int32),
            pltpu.VMEM((GPW,), jnp.int32),
        ],
        compiler_params=pltpu.CompilerParams(needs_layout_passes=False),
        name="cv_agents_ts_sc",
    )
    return f(ids, ts)


def _tc_motion_body(ids_ref, lts_ref, posf_ref, velf_ref, out_ref):
    ids = ids_ref[...]
    lts = lts_ref[...]
    posf = posf_ref[...]
    velf = velf_ref[...]

    gsel = 2 * lts + (ids >> 7)
    px = jnp.zeros((1, A), jnp.float32)
    py = jnp.zeros((1, A), jnp.float32)
    vx = jnp.zeros((1, A), jnp.float32)
    vy = jnp.zeros((1, A), jnp.float32)
    for g in range(2 * L):
        m = gsel == g
        pxg = jnp.concatenate([posf[:, g * 256:g * 256 + 128]] * 2, 1)
        pyg = jnp.concatenate([posf[:, g * 256 + 128:g * 256 + 256]] * 2, 1)
        vxg = jnp.concatenate([velf[:, g * 256:g * 256 + 128]] * 2, 1)
        vyg = jnp.concatenate([velf[:, g * 256 + 128:g * 256 + 256]] * 2, 1)
        px = jnp.where(m, pxg, px)
        py = jnp.where(m, pyg, py)
        vx = jnp.where(m, vxg, vx)
        vy = jnp.where(m, vyg, vy)
    tbl = jnp.concatenate([px.T, py.T, vx.T, vy.T], 1)
    tbl_s = jnp.concatenate([tbl[1:], tbl[255:]], 0)

    r_col = lax.broadcasted_iota(jnp.int32, (NROW, 1), 0)
    b = r_col >> 1
    c = r_col & 1
    p0 = b * 128
    a0 = ((b * 16) * RMAGIC) >> RSHIFT
    thr = (a0 + 1) * PL - p0
    kb0 = p0 - a0 * PL + 1

    a_row = lax.broadcasted_iota(jnp.int32, (1, 256), 1)
    oh_row = (a_row == a0).astype(jnp.bfloat16)
    def _split3(x):
        hi = x.astype(jnp.bfloat16)
        r1 = x - hi.astype(jnp.float32)
        mid = r1.astype(jnp.bfloat16)
        return hi, mid, (r1 - mid.astype(jnp.float32)).astype(jnp.bfloat16)
    h1, m1, l1 = _split3(tbl)
    h2, m2, l2 = _split3(tbl_s)
    rhs = jnp.concatenate([h1, m1, l1, h2, m2, l2], 1)
    y = jnp.dot(oh_row, rhs, preferred_element_type=jnp.float32)
    t1 = y[:, 0:4] + y[:, 4:8] + y[:, 8:12]
    t2 = y[:, 12:16] + y[:, 16:20] + y[:, 20:24]

    cz = c == 0
    p1 = jnp.where(cz, t1[:, 0:1], t1[:, 1:2])
    v1 = jnp.where(cz, t1[:, 2:3], t1[:, 3:4])
    p2 = jnp.where(cz, t2[:, 0:1], t2[:, 1:2])
    v2 = jnp.where(cz, t2[:, 2:3], t2[:, 3:4])

    j_row = lax.broadcasted_iota(jnp.int32, (1, 128), 1)
    s0 = (kb0 + j_row).astype(jnp.float32)
    s1 = s0 - jnp.float32(PL)
    out_ref[...] = jnp.where(j_row < thr, p1 + s0 * v1, p2 + s1 * v2)


def _run_tc(ids2, lts2, posf, velf):
    return pl.pallas_call(
        _tc_motion_body,
        out_shape=jax.ShapeDtypeStruct((NROW, 128), jnp.float32),
        name="cv_motion_tc",
    )(ids2, lts2, posf, velf)


@jax.jit
def _run_all(ids2, lts2, ts1, posf, velf):
    mot = _run_tc(ids2, lts2, posf, velf)
    ag, dts = _run_sc(ids2[0], ts1)
    return mot, ag, dts


def kernel(identities, timesteps, scene_orig, obs_position_sequence,
           obs_velocity_sequence, obs_timestep_sequence, obs_identity_sequence,
           last_obs_positions, last_obs_timesteps, pred_position_sequence,
           pred_velocity_sequence, pred_timestep_sequence, pred_identity_sequence):
    posf = obs_position_sequence[0].reshape(16, 128, 2).transpose(0, 2, 1).reshape(1, -1)
    velf = obs_velocity_sequence[0].reshape(16, 128, 2).transpose(0, 2, 1).reshape(1, -1)
    mot, ag, dts = _run_all(identities, last_obs_timesteps, timesteps[0],
                            posf, velf)
    motion = mot.reshape(2032, 2, 128).transpose(0, 2, 1).reshape(1, N, 2)
    return motion, ag.reshape(1, N), dts
